# Initial kernel scaffold; baseline (speedup 1.0000x reference)
import jax
import jax.numpy as jnp
from jax.experimental import pallas as pl


def kernel(y_patch, yd_patch, idx_k):
    raise NotImplementedError("write your pallas kernel here")


# speedup vs baseline: 1.1547x; 1.1547x over previous
import functools

import jax
import jax.numpy as jnp
from jax import lax
from jax.experimental import pallas as pl
from jax.experimental.pallas import tpu as pltpu
from jax.experimental.pallas import tpu_sc as plsc

D = 128
M = 50000
KNBR = 4
B = 80
NBLK = M // B
NTASK = KNBR * NBLK
NC = 2
NS = 16
NW = NC * NS

_mesh = plsc.VectorSubcoreMesh(core_axis_name="c", subcore_axis_name="s")


@functools.partial(
    pl.kernel,
    out_type=jax.ShapeDtypeStruct((KNBR * D, M), jnp.float32),
    mesh=_mesh,
    compiler_params=pltpu.CompilerParams(
        use_tc_tiling_on_sc=False, needs_layout_passes=False
    ),
    scratch_types=[
        pltpu.VMEM((B,), jnp.int32),
        pltpu.VMEM((B, D), jnp.float32),
        pltpu.VMEM((D, B), jnp.float32),
        pltpu.SemaphoreType.DMA,
    ],
)
def _gather_transpose(yd_hbm, idxt_hbm, out_hbm, idx_v, rows_v, t_v, sem):
    wid = lax.axis_index("s") * NC + lax.axis_index("c")
    viota = lax.iota(jnp.int32, 16)

    @pl.loop(wid, NTASK, step=NW)
    def _task(t):
        kk = t // NBLK
        mm0 = (t % NBLK) * B
        pltpu.sync_copy(idxt_hbm.at[pl.ds(kk * M + mm0, B)], idx_v)
        pltpu.async_copy(yd_hbm.at[idx_v], rows_v, sem).wait()

        @pl.loop(0, (B // 16) * (D // 16))
        def _tile(tt):
            r0 = (tt % (B // 16)) * 16
            f0 = (tt // (B // 16)) * 16
            b = f0 + viota
            for d in range(16):
                a = r0 + ((viota + d) & 15)
                v = plsc.load_gather(rows_v, [a, b])
                plsc.store_scatter(t_v, [b, a], v)

        pltpu.sync_copy(t_v, out_hbm.at[pl.ds(kk * D, D), pl.ds(mm0, B)])


def kernel(y_patch, yd_patch, idx_k):
    del y_patch
    idxt = jnp.transpose(idx_k[0].astype(jnp.int32), (1, 0)).reshape(-1)
    out2 = _gather_transpose(yd_patch, idxt)
    return out2.reshape(1, KNBR * (D // 2), 2 * M)


# speedup vs baseline: 1.5580x; 1.3493x over previous
import functools

import jax
import jax.numpy as jnp
from jax import lax
from jax.experimental import pallas as pl
from jax.experimental.pallas import tpu as pltpu
from jax.experimental.pallas import tpu_sc as plsc

D = 128
M = 50000
KNBR = 4
B = 80
NBLK = M // B
NTASK = KNBR * NBLK
NC = 2
NS = 16
NW = NC * NS
NTPW = -(-NTASK // NW)
NTPW += NTPW % 2

_mesh = plsc.VectorSubcoreMesh(core_axis_name="c", subcore_axis_name="s")


@functools.partial(
    pl.kernel,
    out_type=jax.ShapeDtypeStruct((KNBR * D, M), jnp.float32),
    mesh=_mesh,
    compiler_params=pltpu.CompilerParams(
        use_tc_tiling_on_sc=False, needs_layout_passes=False
    ),
    scratch_types=[
        pltpu.VMEM((NTPW, B), jnp.int32),
        pltpu.VMEM((B, D), jnp.float32),
        pltpu.VMEM((B, D), jnp.float32),
        pltpu.VMEM((D, B), jnp.float32),
        pltpu.VMEM((D, B), jnp.float32),
        pltpu.SemaphoreType.DMA,
        pltpu.SemaphoreType.DMA,
        pltpu.SemaphoreType.DMA,
        pltpu.SemaphoreType.DMA,
    ],
)
def _gather_transpose(yd_hbm, idxw_hbm, out_hbm,
                      idx_all, rows0, rows1, t0, t1,
                      semr0, semr1, semt0, semt1):
    wid = lax.axis_index("s") * NC + lax.axis_index("c")
    last = jnp.where(wid < NTASK - (NTPW - 2) * NW, NTPW - 2, NTPW - 3)
    viota = lax.iota(jnp.int32, 16)
    rowsels = [(viota + d) & 15 for d in range(16)]
    rows = (rows0, rows1)
    ts = (t0, t1)
    semr = (semr0, semr1)
    semt = (semt0, semt1)

    pltpu.sync_copy(idxw_hbm.at[wid], idx_all)

    def start_gather(li, p):
        pltpu.async_copy(
            yd_hbm.at[idx_all.at[jnp.minimum(li, last)]], rows[p], semr[p]
        )

    def wait_rows(p):
        pltpu.make_async_copy(yd_hbm.at[pl.ds(0, B)], rows[p], semr[p]).wait()

    def wait_out(p):
        pltpu.make_async_copy(
            ts[p], out_hbm.at[pl.ds(0, D), pl.ds(0, B)], semt[p]
        ).wait()

    def transpose(p):
        @pl.loop(0, (B // 16) * (D // 16))
        def _tile(tt):
            r0 = (tt % (B // 16)) * 16
            f0 = (tt // (B // 16)) * 16
            b = f0 + viota
            for d in range(16):
                a = r0 + rowsels[d]
                v = plsc.load_gather(rows[p], [a, b])
                plsc.store_scatter(ts[p], [b, a], v)

    def start_out(li, p):
        t = wid + jnp.minimum(li, last) * NW
        kk = t // NBLK
        mm0 = (t % NBLK) * B
        pltpu.async_copy(
            ts[p], out_hbm.at[pl.ds(kk * D, D), pl.ds(mm0, B)], semt[p]
        )

    def step(li, p, start_next=True, wait_t=True):
        if start_next:
            start_gather(li + 1, 1 - p)
        if wait_t:
            wait_out(p)
        wait_rows(p)
        transpose(p)
        start_out(li, p)

    start_gather(0, 0)
    step(0, 0, wait_t=False)
    step(1, 1, wait_t=False)

    @pl.loop(2, NTPW - 2, step=2)
    def _main(i):
        step(i, 0)
        step(i + 1, 1)

    step(NTPW - 2, 0)
    step(NTPW - 1, 1, start_next=False)
    wait_out(0)
    wait_out(1)


def kernel(y_patch, yd_patch, idx_k):
    del y_patch
    idxt = jnp.transpose(idx_k[0].astype(jnp.int32), (1, 0)).reshape(-1)
    tasks = jnp.pad(idxt.reshape(NTASK, B), ((0, NW * NTPW - NTASK), (0, 0)))
    idxw = jnp.transpose(tasks.reshape(NTPW, NW, B), (1, 0, 2))
    out2 = _gather_transpose(yd_patch, idxw)
    return out2.reshape(1, KNBR * (D // 2), 2 * M)
